# Initial kernel scaffold; baseline (speedup 1.0000x reference)
import jax
import jax.numpy as jnp
from jax.experimental import pallas as pl


def kernel(x, W):
    raise NotImplementedError("write your pallas kernel here")


# speedup vs baseline: 4.1826x; 4.1826x over previous
import functools

import jax
import jax.numpy as jnp
from jax import lax
from jax.experimental import pallas as pl
from jax.experimental.pallas import tpu as pltpu
from jax.experimental.pallas import tpu_sc as plsc

V = 27
D = 27
L = 16
NC = 2
NS = 16
NW = NC * NS
CH = 1024


def _sc_gather(x_flat, w_flat):
    b = x_flat.shape[0]
    b_per_w = b // NW
    n_chunks = b_per_w // CH
    mesh = plsc.VectorSubcoreMesh(core_axis_name="c", subcore_axis_name="s")

    @functools.partial(
        pl.kernel,
        mesh=mesh,
        compiler_params=pltpu.CompilerParams(needs_layout_passes=False),
        out_type=jax.ShapeDtypeStruct((b * D,), jnp.float32),
        scratch_types=[
            pltpu.VMEM((V * D,), jnp.float32),
            pltpu.VMEM((CH,), jnp.int32),
            pltpu.VMEM((CH * D,), jnp.float32),
        ],
    )
    def k(x_hbm, w_hbm, out_hbm, w_v, idx_v, out_v):
        wid = lax.axis_index("s") * NC + lax.axis_index("c")
        base = wid * b_per_w
        pltpu.sync_copy(w_hbm, w_v)
        r27 = lax.broadcasted_iota(jnp.int32, (L,), 0) * D

        def chunk_body(c, carry):
            r0 = base + c * CH
            pltpu.sync_copy(x_hbm.at[pl.ds(r0, CH)], idx_v)

            def blk_body(j, carry2):
                idx = idx_v[pl.ds(j * L, L)]
                src = idx * D
                dst = r27 + j * (L * D)
                for kcol in range(D):
                    vals = plsc.load_gather(w_v, [src + kcol])
                    plsc.store_scatter(out_v, [dst + kcol], vals)
                return carry2

            lax.fori_loop(0, CH // L, blk_body, 0)
            pltpu.sync_copy(out_v, out_hbm.at[pl.ds(r0 * D, CH * D)])
            return carry

        lax.fori_loop(0, n_chunks, chunk_body, 0)

    return k(x_flat, w_flat)


def kernel(x, W):
    b = x.size
    x_flat = x.reshape(b).astype(jnp.int32)
    out = _sc_gather(x_flat, W.astype(jnp.float32).reshape(V * D))
    return out.reshape(x.shape + (D,))


# speedup vs baseline: 4.7814x; 1.1432x over previous
import functools

import jax
import jax.numpy as jnp
from jax import lax
from jax.experimental import pallas as pl
from jax.experimental.pallas import tpu as pltpu
from jax.experimental.pallas import tpu_sc as plsc

V = 27
D = 27
L = 16
NC = 2
NS = 16
NW = NC * NS
CH = 1600
UNROLL = 2


def _sc_gather(x_flat, w_flat):
    b = x_flat.shape[0]
    b_per_w = b // NW
    n_chunks = b_per_w // CH
    assert n_chunks % 2 == 0
    mesh = plsc.VectorSubcoreMesh(core_axis_name="c", subcore_axis_name="s")

    @functools.partial(
        pl.kernel,
        mesh=mesh,
        compiler_params=pltpu.CompilerParams(needs_layout_passes=False),
        out_type=jax.ShapeDtypeStruct((b * D,), jnp.float32),
        scratch_types=[
            pltpu.VMEM((V * D,), jnp.float32),
            pltpu.VMEM((CH,), jnp.int32),
            pltpu.VMEM((CH,), jnp.int32),
            pltpu.VMEM((CH * D,), jnp.float32),
            pltpu.VMEM((CH * D,), jnp.float32),
            pltpu.SemaphoreType.DMA,
            pltpu.SemaphoreType.DMA,
            pltpu.SemaphoreType.DMA,
            pltpu.SemaphoreType.DMA,
        ],
    )
    def k(x_hbm, w_hbm, out_hbm, w_v, idx_a, idx_b, out_a, out_b,
          sin_a, sin_b, sout_a, sout_b):
        wid = lax.axis_index("s") * NC + lax.axis_index("c")
        base = wid * b_per_w
        pltpu.sync_copy(w_hbm, w_v)

        idx_bufs = (idx_a, idx_b)
        out_bufs = (out_a, out_b)
        sins = (sin_a, sin_b)
        souts = (sout_a, sout_b)

        def in_copy(c, buf):
            r0 = base + c * CH
            return pltpu.make_async_copy(
                x_hbm.at[pl.ds(r0, CH)], idx_bufs[buf], sins[buf])

        def out_copy(c, buf):
            r0 = base + c * CH
            return pltpu.make_async_copy(
                out_bufs[buf], out_hbm.at[pl.ds(r0 * D, CH * D)], souts[buf])

        in_copy(0, 0).start()
        in_copy(1, 1).start()

        r27 = lax.broadcasted_iota(jnp.int32, (L,), 0) * D

        def compute(idx_ref, out_ref):
            @plsc.parallel_loop(0, CH // L, unroll=UNROLL)
            def blk(j):
                idx = idx_ref[pl.ds(j * L, L)]
                src = idx * D
                dst = r27 + j * (L * D)
                for kcol in range(D):
                    vals = plsc.load_gather(w_v, [src + kcol])
                    plsc.store_scatter(out_ref, [dst + kcol], vals)

        def pair_body(p, carry):
            for buf in range(2):
                c = p * 2 + buf
                in_copy(c, buf).wait()

                @pl.when(c >= 2)
                def _wait_out():
                    out_copy(c - 2, buf).wait()

                compute(idx_bufs[buf], out_bufs[buf])
                out_copy(c, buf).start()

                @pl.when(c + 2 < n_chunks)
                def _prefetch():
                    in_copy(c + 2, buf).start()
            return carry

        lax.fori_loop(0, n_chunks // 2, pair_body, 0)
        out_copy(n_chunks - 2, 0).wait()
        out_copy(n_chunks - 1, 1).wait()

    return k(x_flat, w_flat)


def kernel(x, W):
    b = x.size
    x_flat = x.reshape(b).astype(jnp.int32)
    out = _sc_gather(x_flat, W.astype(jnp.float32).reshape(V * D))
    return out.reshape(x.shape + (D,))


# speedup vs baseline: 4.9790x; 1.0413x over previous
import functools

import jax
import jax.numpy as jnp
from jax import lax
from jax.experimental import pallas as pl
from jax.experimental.pallas import tpu as pltpu
from jax.experimental.pallas import tpu_sc as plsc

V = 27
D = 27
L = 16
NC = 2
NS = 16
NW = NC * NS
CH = 1600
UNROLL = 2


def _sc_gather(x_flat, w_flat):
    b = x_flat.shape[0]
    b_per_w = b // NW
    n_chunks = b_per_w // CH
    assert n_chunks % 2 == 0
    mesh = plsc.VectorSubcoreMesh(core_axis_name="c", subcore_axis_name="s")

    @functools.partial(
        pl.kernel,
        mesh=mesh,
        compiler_params=pltpu.CompilerParams(needs_layout_passes=False),
        out_type=jax.ShapeDtypeStruct((b * D,), jnp.float32),
        scratch_types=[
            pltpu.VMEM((V * D,), jnp.float32),
            pltpu.VMEM((CH,), jnp.int32),
            pltpu.VMEM((CH,), jnp.int32),
            pltpu.VMEM((CH * D,), jnp.float32),
            pltpu.VMEM((CH * D,), jnp.float32),
            pltpu.SemaphoreType.DMA,
            pltpu.SemaphoreType.DMA,
            pltpu.SemaphoreType.DMA,
            pltpu.SemaphoreType.DMA,
        ],
    )
    def k(x_hbm, w_hbm, out_hbm, w_v, idx_a, idx_b, out_a, out_b,
          sin_a, sin_b, sout_a, sout_b):
        wid = lax.axis_index("s") * NC + lax.axis_index("c")
        base = wid * b_per_w
        pltpu.sync_copy(w_hbm, w_v)

        idx_bufs = (idx_a, idx_b)
        out_bufs = (out_a, out_b)
        sins = (sin_a, sin_b)
        souts = (sout_a, sout_b)

        def in_copy(c, buf):
            r0 = base + c * CH
            return pltpu.make_async_copy(
                x_hbm.at[pl.ds(r0, CH)], idx_bufs[buf], sins[buf])

        def out_copy(c, buf):
            r0 = base + c * CH
            return pltpu.make_async_copy(
                out_bufs[buf], out_hbm.at[pl.ds(r0 * D, CH * D)], souts[buf])

        in_copy(0, 0).start()
        in_copy(1, 1).start()

        r27 = lax.broadcasted_iota(jnp.int32, (L,), 0) * D

        def compute(idx_ref, out_ref):
            @plsc.parallel_loop(0, CH // L, unroll=UNROLL)
            def blk(j):
                idx = idx_ref[pl.ds(j * L, L)]
                src = idx * D
                dst = r27 + j * (L * D)
                vals = [plsc.load_gather(w_v, [src + kcol])
                        for kcol in range(D)]
                for kcol in range(D):
                    plsc.store_scatter(out_ref, [dst + kcol], vals[kcol])

        def pair_body(p, carry):
            for buf in range(2):
                c = p * 2 + buf
                in_copy(c, buf).wait()

                @pl.when(c >= 2)
                def _wait_out():
                    out_copy(c - 2, buf).wait()

                compute(idx_bufs[buf], out_bufs[buf])
                out_copy(c, buf).start()

                @pl.when(c + 2 < n_chunks)
                def _prefetch():
                    in_copy(c + 2, buf).start()
            return carry

        lax.fori_loop(0, n_chunks // 2, pair_body, 0)
        out_copy(n_chunks - 2, 0).wait()
        out_copy(n_chunks - 1, 1).wait()

    return k(x_flat, w_flat)


def kernel(x, W):
    b = x.size
    x_flat = x.reshape(b).astype(jnp.int32)
    out = _sc_gather(x_flat, W.astype(jnp.float32).reshape(V * D))
    return out.reshape(x.shape + (D,))


# speedup vs baseline: 39.2342x; 7.8800x over previous
import functools

import jax
import jax.numpy as jnp
from jax import lax
from jax.experimental import pallas as pl
from jax.experimental.pallas import tpu as pltpu
from jax.experimental.pallas import tpu_sc as plsc

V = 27
D = 27
L = 16
NC = 2
NS = 16
NW = NC * NS

NI = 16384
NJ = 50
JT = 7
IB_ALL = NI // 128
IB_PW = IB_ALL // NW
ROWS_PW = 128 * IB_PW * NJ
STG = D * 1024
KSTRIDE = JT * 131072
OUT_WORDS = D * KSTRIDE
N_PH = JT * IB_PW


def _sc_gather(x_flat, w_flat):
    mesh = plsc.VectorSubcoreMesh(core_axis_name="c", subcore_axis_name="s")

    @functools.partial(
        pl.kernel,
        mesh=mesh,
        compiler_params=pltpu.CompilerParams(needs_layout_passes=False),
        out_type=jax.ShapeDtypeStruct((OUT_WORDS,), jnp.float32),
        scratch_types=[
            pltpu.VMEM((V * D,), jnp.float32),
            pltpu.VMEM((ROWS_PW,), jnp.int32),
            pltpu.VMEM((STG,), jnp.float32),
            pltpu.VMEM((STG,), jnp.float32),
            pltpu.SemaphoreType.DMA,
            pltpu.SemaphoreType.DMA,
        ],
    )
    def k(x_hbm, w_hbm, out_hbm, w_v, xb_v, stg_a, stg_b, sem_a, sem_b):
        wid = lax.axis_index("s") * NC + lax.axis_index("c")
        pltpu.sync_copy(w_hbm, w_v)
        pltpu.sync_copy(x_hbm.at[pl.ds(wid * ROWS_PW, ROWS_PW)], xb_v)

        stgs = (stg_a, stg_b)
        sems = (sem_a, sem_b)
        str50 = lax.broadcasted_iota(jnp.int32, (L,), 0) * NJ
        ib0 = wid * IB_PW

        def out_dma(ph, buf, kk):
            jt = ph >> 2
            ib = ib0 + (ph & 3)
            dst = kk * KSTRIDE + jt * 131072 + ib * 1024
            return pltpu.make_async_copy(
                stgs[buf].at[pl.ds(kk * 1024, 1024)],
                out_hbm.at[pl.ds(dst, 1024)],
                sems[buf])

        def compute(ph, buf):
            jt = ph >> 2
            ib = ph & 3
            jrc = jnp.where(jt == JT - 1, NJ - 8 * (JT - 1), 8)
            stg = stgs[buf]

            @plsc.parallel_loop(0, 8)
            def s_loop(s):
                base_i = (ib * 128 + s * 16) * NJ + jt * 8

                @plsc.parallel_loop(0, jrc)
                def jr_loop(jr):
                    xg = plsc.load_gather(xb_v, [str50 + (base_i + jr)])
                    wa = xg * D
                    sj = jr * 128 + s * 16
                    for kk in range(D):
                        wv = plsc.load_gather(w_v, [wa + kk])
                        stg[pl.ds(kk * 1024 + sj, L)] = wv

        def pair_body(p2, carry):
            for buf in range(2):
                ph = p2 * 2 + buf

                @pl.when(ph >= 2)
                def _drain():
                    def wbody(kk, c2):
                        out_dma(ph - 2, buf, kk).wait()
                        return c2
                    lax.fori_loop(0, D, wbody, 0)

                compute(ph, buf)

                def sbody(kk, c2):
                    out_dma(ph, buf, kk).start()
                    return c2
                lax.fori_loop(0, D, sbody, 0)
            return carry

        lax.fori_loop(0, N_PH // 2, pair_body, 0)

        def wlast(kk, c2):
            out_dma(N_PH - 2, 0, kk).wait()
            out_dma(N_PH - 1, 1, kk).wait()
            return c2
        lax.fori_loop(0, D, wlast, 0)

    return k(x_flat, w_flat)


def kernel(x, W):
    assert x.shape == (NI, NJ) and W.shape == (V, D)
    x_flat = x.reshape(NI * NJ).astype(jnp.int32)
    out1 = _sc_gather(x_flat, W.astype(jnp.float32).reshape(V * D))
    a = out1.reshape(D, JT, 128, 8, 128)
    b = jnp.transpose(a, (2, 4, 1, 3, 0))
    c = b.reshape(NI, 8 * JT, D)
    return c[:, :NJ, :]
